# Initial kernel scaffold; baseline (speedup 1.0000x reference)
import jax
import jax.numpy as jnp
from jax.experimental import pallas as pl


def kernel(x, W1l, W1r, att1, b1, a1, W2l, W2r, att2, b2, a2):
    raise NotImplementedError("write your pallas kernel here")


# speedup vs baseline: 12.9480x; 12.9480x over previous
import jax
import jax.numpy as jnp
from jax.experimental import pallas as pl
from jax.experimental.pallas import tpu as pltpu
from jax.experimental.pallas import tpu_sc as plsc

N = 10000
IN_CH = 128
REP = 128
HEADS = 8
K = 6
SLOTS = 8
BR1 = 200
BR2 = 200
BR4 = 400

_HIGH = jax.lax.Precision.HIGHEST
NEG = -1e30


def _leaky(e):
    return jnp.where(e > 0, e, 0.2 * e)


def _topk_body(x_ref, xT_ref, nbr_ref, mb_ref, s_ref):
    i0 = pl.program_id(0) * BR1
    s = jnp.dot(x_ref[...], xT_ref[...],
                preferred_element_type=jnp.float32)
    col = jax.lax.broadcasted_iota(jnp.int32, (BR1, N), 1)
    s_ref[...] = s
    rows = i0 + jax.lax.broadcasted_iota(jnp.int32, (BR1, 1), 0)
    vals, idxs = [], []
    for _ in range(K):
        s = s_ref[...]
        m = jnp.max(s, axis=1, keepdims=True)
        cand = jnp.where(s == m, col, N)
        sel = jnp.min(cand, axis=1, keepdims=True)
        vals.append(m)
        idxs.append(sel)
        s_ref[...] = jnp.where(col == sel, -jnp.inf, s)
    v6 = jnp.concatenate(vals, axis=1)
    i6 = jnp.concatenate(idxs, axis=1)
    valid = (i6 > rows) & (v6 != 0.0)
    nbr_ref[...] = jnp.concatenate(
        [jnp.where(valid, i6, rows), rows, rows], axis=1)
    zeros = jnp.zeros((BR1, 1), jnp.float32)
    mb_ref[...] = jnp.concatenate(
        [jnp.where(valid, 0.0, NEG), zeros, zeros + NEG], axis=1)


def _topk(x, xT):
    return pl.pallas_call(
        _topk_body,
        grid=(N // BR1,),
        in_specs=[
            pl.BlockSpec((BR1, IN_CH), lambda i: (i, 0)),
            pl.BlockSpec((IN_CH, N), lambda i: (0, 0)),
        ],
        out_specs=[
            pl.BlockSpec((BR1, SLOTS), lambda i: (i, 0)),
            pl.BlockSpec((BR1, SLOTS), lambda i: (i, 0)),
        ],
        out_shape=[
            jax.ShapeDtypeStruct((N, SLOTS), jnp.int32),
            jax.ShapeDtypeStruct((N, SLOTS), jnp.float32),
        ],
        scratch_shapes=[pltpu.VMEM((BR1, N), jnp.float32)],
        compiler_params=pltpu.CompilerParams(
            dimension_semantics=("parallel",)),
    )(x, xT)


_GW = 128


def _gather(table, idx_flat):
    total = idx_flat.shape[1]
    mesh = plsc.VectorSubcoreMesh(core_axis_name="core",
                                  subcore_axis_name="subcore")

    @pl.kernel(
        out_type=jax.ShapeDtypeStruct((total, table.shape[1]), table.dtype),
        mesh=mesh)
    def kern(tab_hbm, i_hbm, o_hbm):
        def body(i_vmem, o_vmem):
            pltpu.sync_copy(tab_hbm.at[i_vmem.at[0]], o_vmem)

        pltpu.emit_pipeline(
            body,
            grid=(total // _GW,),
            in_specs=[pl.BlockSpec((1, _GW), index_map=lambda i: (0, i))],
            out_specs=[pl.BlockSpec((_GW, table.shape[1]),
                                    index_map=lambda i: (i, 0))],
            core_axis_name="subcore",
            dimension_semantics=(pltpu.PARALLEL,),
        )(i_hbm, o_hbm)

    return kern(table, idx_flat)


def _layer1_body(x_ref, xg_ref, mb_ref, W1l_ref, W1r_ref, att_ref, b1_ref,
                 a1_ref, W2l_ref, W2r_ref, yl_ref, yr_ref):
    xr = jnp.dot(x_ref[...], W1r_ref[...], precision=_HIGH,
                 preferred_element_type=jnp.float32)
    att = att_ref[...]
    alphas, xlgs = [], []
    for s in range(SLOTS):
        xlg = jnp.dot(xg_ref[s], W1l_ref[...], precision=_HIGH,
                      preferred_element_type=jnp.float32)
        e = _leaky(xlg + xr) * att
        al = e.reshape(BR2, HEADS, REP).sum(-1)
        al = al + mb_ref[:, s:s + 1]
        xlgs.append(xlg)
        alphas.append(al)
    m = alphas[0]
    for s in range(1, SLOTS):
        m = jnp.maximum(m, alphas[s])
    exs = [jnp.exp(al - m) for al in alphas]
    den = exs[0]
    for s in range(1, SLOTS):
        den = den + exs[s]
    acc = jnp.zeros((BR2, HEADS * REP), jnp.float32)
    for s in range(SLOTS):
        w = exs[s] / den
        wf = jnp.broadcast_to(w[:, :, None], (BR2, HEADS, REP))
        acc = acc + wf.reshape(BR2, HEADS * REP) * xlgs[s]
    h = acc + b1_ref[...]
    h = jnp.where(h > 0, h, a1_ref[0, 0] * h)
    yl_ref[...] = jnp.dot(h, W2l_ref[...], precision=_HIGH,
                          preferred_element_type=jnp.float32)
    yr_ref[...] = jnp.dot(h, W2r_ref[...], precision=_HIGH,
                          preferred_element_type=jnp.float32)


def _layer1(x, xg, mb, W1l, W1r, att1, b1, a1, W2l, W2r):
    D1 = HEADS * REP
    return pl.pallas_call(
        _layer1_body,
        grid=(N // BR2,),
        in_specs=[
            pl.BlockSpec((BR2, IN_CH), lambda i: (i, 0)),
            pl.BlockSpec((SLOTS, BR2, IN_CH), lambda i: (0, i, 0)),
            pl.BlockSpec((BR2, SLOTS), lambda i: (i, 0)),
            pl.BlockSpec((IN_CH, D1), lambda i: (0, 0)),
            pl.BlockSpec((IN_CH, D1), lambda i: (0, 0)),
            pl.BlockSpec((1, D1), lambda i: (0, 0)),
            pl.BlockSpec((1, D1), lambda i: (0, 0)),
            pl.BlockSpec((1, 1), lambda i: (0, 0)),
            pl.BlockSpec((D1, REP), lambda i: (0, 0)),
            pl.BlockSpec((D1, REP), lambda i: (0, 0)),
        ],
        out_specs=[
            pl.BlockSpec((BR2, REP), lambda i: (i, 0)),
            pl.BlockSpec((BR2, REP), lambda i: (i, 0)),
        ],
        out_shape=[
            jax.ShapeDtypeStruct((N, REP), jnp.float32),
            jax.ShapeDtypeStruct((N, REP), jnp.float32),
        ],
        compiler_params=pltpu.CompilerParams(
            dimension_semantics=("parallel",)),
    )(x, xg, mb, W1l, W1r, att1, b1, a1, W2l, W2r)


def _layer2_body(ylg_ref, yr_ref, mb_ref, att_ref, b2_ref, a2_ref, o_ref):
    yr = yr_ref[...]
    att = att_ref[...]
    alphas = []
    for s in range(SLOTS):
        e = _leaky(ylg_ref[s] + yr) * att
        al = e.sum(-1, keepdims=True) + mb_ref[:, s:s + 1]
        alphas.append(al)
    m = alphas[0]
    for s in range(1, SLOTS):
        m = jnp.maximum(m, alphas[s])
    exs = [jnp.exp(al - m) for al in alphas]
    den = exs[0]
    for s in range(1, SLOTS):
        den = den + exs[s]
    acc = jnp.zeros((BR4, REP), jnp.float32)
    for s in range(SLOTS):
        acc = acc + (exs[s] / den) * ylg_ref[s]
    out = acc + b2_ref[...]
    o_ref[...] = jnp.where(out > 0, out, a2_ref[0, 0] * out)


def _layer2(ylg, yr, mb, att2, b2, a2):
    return pl.pallas_call(
        _layer2_body,
        grid=(N // BR4,),
        in_specs=[
            pl.BlockSpec((SLOTS, BR4, REP), lambda i: (0, i, 0)),
            pl.BlockSpec((BR4, REP), lambda i: (i, 0)),
            pl.BlockSpec((BR4, SLOTS), lambda i: (i, 0)),
            pl.BlockSpec((1, REP), lambda i: (0, 0)),
            pl.BlockSpec((1, REP), lambda i: (0, 0)),
            pl.BlockSpec((1, 1), lambda i: (0, 0)),
        ],
        out_specs=pl.BlockSpec((BR4, REP), lambda i: (i, 0)),
        out_shape=jax.ShapeDtypeStruct((N, REP), jnp.float32),
        compiler_params=pltpu.CompilerParams(
            dimension_semantics=("parallel",)),
    )(ylg, yr, mb, att2, b2, a2)


def kernel(x, W1l, W1r, att1, b1, a1, W2l, W2r, att2, b2, a2):
    xT = x.T
    nbr, mb = _topk(x, xT)
    idx = nbr.T.reshape(1, SLOTS * N)
    xg = _gather(x, idx).reshape(SLOTS, N, IN_CH)
    yl, yr = _layer1(x, xg, mb,
                     W1l, W1r,
                     att1.reshape(1, HEADS * REP),
                     b1.reshape(1, HEADS * REP),
                     jnp.reshape(a1, (1, 1)),
                     W2l, W2r)
    ylg = _gather(yl, idx).reshape(SLOTS, N, REP)
    out = _layer2(ylg, yr, mb,
                  att2.reshape(1, REP),
                  b2.reshape(1, REP),
                  jnp.reshape(a2, (1, 1)))
    return out


# speedup vs baseline: 14.8834x; 1.1495x over previous
import jax
import jax.numpy as jnp
from jax.experimental import pallas as pl
from jax.experimental.pallas import tpu as pltpu
from jax.experimental.pallas import tpu_sc as plsc

N = 10000
IN_CH = 128
REP = 128
HEADS = 8
K = 6
SLOTS = 8
BR1 = 200
BR2 = 200
BR4 = 400

NEG = -1e30


def _leaky(e):
    return jnp.where(e > 0, e, 0.2 * e)


def _topk_body(x_ref, xT_ref, nbr_ref, mb_ref, s_ref):
    i0 = pl.program_id(0) * BR1
    s = jnp.dot(x_ref[...], xT_ref[...],
                preferred_element_type=jnp.float32)
    col = jax.lax.broadcasted_iota(jnp.int32, (BR1, N), 1)
    s_ref[...] = s
    rows = i0 + jax.lax.broadcasted_iota(jnp.int32, (BR1, 1), 0)
    vals, idxs = [], []
    for _ in range(K):
        s = s_ref[...]
        m = jnp.max(s, axis=1, keepdims=True)
        cand = jnp.where(s == m, col, N)
        sel = jnp.min(cand, axis=1, keepdims=True)
        vals.append(m)
        idxs.append(sel)
        s_ref[...] = jnp.where(col == sel, -jnp.inf, s)
    v6 = jnp.concatenate(vals, axis=1)
    i6 = jnp.concatenate(idxs, axis=1)
    valid = (i6 > rows) & (v6 != 0.0)
    nbr_ref[...] = jnp.concatenate(
        [jnp.where(valid, i6, rows), rows, rows], axis=1)
    zeros = jnp.zeros((BR1, 1), jnp.float32)
    mb_ref[...] = jnp.concatenate(
        [jnp.where(valid, 0.0, NEG), zeros, zeros + NEG], axis=1)


def _topk(x, xT):
    return pl.pallas_call(
        _topk_body,
        grid=(N // BR1,),
        in_specs=[
            pl.BlockSpec((BR1, IN_CH), lambda i: (i, 0)),
            pl.BlockSpec((IN_CH, N), lambda i: (0, 0)),
        ],
        out_specs=[
            pl.BlockSpec((BR1, SLOTS), lambda i: (i, 0)),
            pl.BlockSpec((BR1, SLOTS), lambda i: (i, 0)),
        ],
        out_shape=[
            jax.ShapeDtypeStruct((N, SLOTS), jnp.int32),
            jax.ShapeDtypeStruct((N, SLOTS), jnp.float32),
        ],
        scratch_shapes=[pltpu.VMEM((BR1, N), jnp.float32)],
        compiler_params=pltpu.CompilerParams(
            dimension_semantics=("parallel",)),
    )(x, xT)


_GW = 128


def _gather(table, idx_flat):
    total = idx_flat.shape[1]
    mesh = plsc.VectorSubcoreMesh(core_axis_name="core",
                                  subcore_axis_name="subcore")

    @pl.kernel(
        out_type=jax.ShapeDtypeStruct((total, table.shape[1]), table.dtype),
        mesh=mesh)
    def kern(tab_hbm, i_hbm, o_hbm):
        def body(i_vmem, o_vmem):
            pltpu.sync_copy(tab_hbm.at[i_vmem.at[0]], o_vmem)

        pltpu.emit_pipeline(
            body,
            grid=(total // _GW,),
            in_specs=[pl.BlockSpec((1, _GW), index_map=lambda i: (0, i))],
            out_specs=[pl.BlockSpec((_GW, table.shape[1]),
                                    index_map=lambda i: (i, 0))],
            core_axis_name="subcore",
            dimension_semantics=(pltpu.PARALLEL,),
        )(i_hbm, o_hbm)

    return kern(table, idx_flat)


def _layer1_body(x_ref, xg_ref, mb_ref, W1l_ref, W1r_ref, att_ref, b1_ref,
                 a1_ref, W2l_ref, W2r_ref, yl_ref, yr_ref):
    xr = jnp.dot(x_ref[...], W1r_ref[...],
                 preferred_element_type=jnp.float32)
    att = att_ref[...]
    alphas, xlgs = [], []
    for s in range(SLOTS):
        xlg = jnp.dot(xg_ref[s], W1l_ref[...],
                      preferred_element_type=jnp.float32)
        e = _leaky(xlg + xr) * att
        al = e.reshape(BR2, HEADS, REP).sum(-1)
        al = al + mb_ref[:, s:s + 1]
        xlgs.append(xlg)
        alphas.append(al)
    m = alphas[0]
    for s in range(1, SLOTS):
        m = jnp.maximum(m, alphas[s])
    exs = [jnp.exp(al - m) for al in alphas]
    den = exs[0]
    for s in range(1, SLOTS):
        den = den + exs[s]
    acc = jnp.zeros((BR2, HEADS * REP), jnp.float32)
    for s in range(SLOTS):
        w = exs[s] / den
        wf = jnp.broadcast_to(w[:, :, None], (BR2, HEADS, REP))
        acc = acc + wf.reshape(BR2, HEADS * REP) * xlgs[s]
    h = acc + b1_ref[...]
    h = jnp.where(h > 0, h, a1_ref[0, 0] * h)
    yl_ref[...] = jnp.dot(h, W2l_ref[...],
                          preferred_element_type=jnp.float32)
    yr_ref[...] = jnp.dot(h, W2r_ref[...],
                          preferred_element_type=jnp.float32)


def _layer1(x, xg, mb, W1l, W1r, att1, b1, a1, W2l, W2r):
    D1 = HEADS * REP
    return pl.pallas_call(
        _layer1_body,
        grid=(N // BR2,),
        in_specs=[
            pl.BlockSpec((BR2, IN_CH), lambda i: (i, 0)),
            pl.BlockSpec((SLOTS, BR2, IN_CH), lambda i: (0, i, 0)),
            pl.BlockSpec((BR2, SLOTS), lambda i: (i, 0)),
            pl.BlockSpec((IN_CH, D1), lambda i: (0, 0)),
            pl.BlockSpec((IN_CH, D1), lambda i: (0, 0)),
            pl.BlockSpec((1, D1), lambda i: (0, 0)),
            pl.BlockSpec((1, D1), lambda i: (0, 0)),
            pl.BlockSpec((1, 1), lambda i: (0, 0)),
            pl.BlockSpec((D1, REP), lambda i: (0, 0)),
            pl.BlockSpec((D1, REP), lambda i: (0, 0)),
        ],
        out_specs=[
            pl.BlockSpec((BR2, REP), lambda i: (i, 0)),
            pl.BlockSpec((BR2, REP), lambda i: (i, 0)),
        ],
        out_shape=[
            jax.ShapeDtypeStruct((N, REP), jnp.float32),
            jax.ShapeDtypeStruct((N, REP), jnp.float32),
        ],
        compiler_params=pltpu.CompilerParams(
            dimension_semantics=("parallel",)),
    )(x, xg, mb, W1l, W1r, att1, b1, a1, W2l, W2r)


def _layer2_body(ylg_ref, yr_ref, mb_ref, att_ref, b2_ref, a2_ref, o_ref):
    yr = yr_ref[...]
    att = att_ref[...]
    alphas = []
    for s in range(SLOTS):
        e = _leaky(ylg_ref[s] + yr) * att
        al = e.sum(-1, keepdims=True) + mb_ref[:, s:s + 1]
        alphas.append(al)
    m = alphas[0]
    for s in range(1, SLOTS):
        m = jnp.maximum(m, alphas[s])
    exs = [jnp.exp(al - m) for al in alphas]
    den = exs[0]
    for s in range(1, SLOTS):
        den = den + exs[s]
    acc = jnp.zeros((BR4, REP), jnp.float32)
    for s in range(SLOTS):
        acc = acc + (exs[s] / den) * ylg_ref[s]
    out = acc + b2_ref[...]
    o_ref[...] = jnp.where(out > 0, out, a2_ref[0, 0] * out)


def _layer2(ylg, yr, mb, att2, b2, a2):
    return pl.pallas_call(
        _layer2_body,
        grid=(N // BR4,),
        in_specs=[
            pl.BlockSpec((SLOTS, BR4, REP), lambda i: (0, i, 0)),
            pl.BlockSpec((BR4, REP), lambda i: (i, 0)),
            pl.BlockSpec((BR4, SLOTS), lambda i: (i, 0)),
            pl.BlockSpec((1, REP), lambda i: (0, 0)),
            pl.BlockSpec((1, REP), lambda i: (0, 0)),
            pl.BlockSpec((1, 1), lambda i: (0, 0)),
        ],
        out_specs=pl.BlockSpec((BR4, REP), lambda i: (i, 0)),
        out_shape=jax.ShapeDtypeStruct((N, REP), jnp.float32),
        compiler_params=pltpu.CompilerParams(
            dimension_semantics=("parallel",)),
    )(ylg, yr, mb, att2, b2, a2)


def kernel(x, W1l, W1r, att1, b1, a1, W2l, W2r, att2, b2, a2):
    xT = x.T
    nbr, mb = _topk(x, xT)
    idx = nbr.T.reshape(1, SLOTS * N)
    xg = _gather(x, idx).reshape(SLOTS, N, IN_CH)
    yl, yr = _layer1(x, xg, mb,
                     W1l, W1r,
                     att1.reshape(1, HEADS * REP),
                     b1.reshape(1, HEADS * REP),
                     jnp.reshape(a1, (1, 1)),
                     W2l, W2r)
    ylg = _gather(yl, idx).reshape(SLOTS, N, REP)
    out = _layer2(ylg, yr, mb,
                  att2.reshape(1, REP),
                  b2.reshape(1, REP),
                  jnp.reshape(a2, (1, 1)))
    return out


# speedup vs baseline: 18.5893x; 1.2490x over previous
import jax
import jax.numpy as jnp
from jax.experimental import pallas as pl
from jax.experimental.pallas import tpu as pltpu
from jax.experimental.pallas import tpu_sc as plsc

N = 10000
IN_CH = 128
REP = 128
HEADS = 8
K = 6
SLOTS = 8
BR1 = 200
BR2 = 200
BR4 = 400

NEG = -1e30


def _leaky(e):
    return jnp.where(e > 0, e, 0.2 * e)


def _topk_body(x_ref, xT_ref, nbr_ref, mb_ref, s_ref):
    i0 = pl.program_id(0) * BR1
    sm = jnp.dot(x_ref[...], xT_ref[...],
                 preferred_element_type=jnp.float32)
    col = jax.lax.broadcasted_iota(jnp.int32, (BR1, N), 1)
    rows = i0 + jax.lax.broadcasted_iota(jnp.int32, (BR1, 1), 0)
    vals, sels = [], []
    for k in range(K):
        if k > 0:
            sm = jnp.where(col == sels[k - 1], -jnp.inf, s_ref[...])
        m = jnp.max(sm, axis=1, keepdims=True)
        cand = jnp.where(sm == m, col, N)
        sel = jnp.min(cand, axis=1, keepdims=True)
        if k < K - 1:
            s_ref[...] = sm
        vals.append(m)
        sels.append(sel)
    v6 = jnp.concatenate(vals, axis=1)
    i6 = jnp.concatenate(sels, axis=1)
    valid = (i6 > rows) & (v6 != 0.0)
    nbr_ref[...] = jnp.concatenate(
        [jnp.where(valid, i6, rows), rows, rows], axis=1)
    zeros = jnp.zeros((BR1, 1), jnp.float32)
    mb_ref[...] = jnp.concatenate(
        [jnp.where(valid, 0.0, NEG), zeros, zeros + NEG], axis=1)


def _topk(x, xT):
    return pl.pallas_call(
        _topk_body,
        grid=(N // BR1,),
        in_specs=[
            pl.BlockSpec((BR1, IN_CH), lambda i: (i, 0)),
            pl.BlockSpec((IN_CH, N), lambda i: (0, 0)),
        ],
        out_specs=[
            pl.BlockSpec((BR1, SLOTS), lambda i: (i, 0)),
            pl.BlockSpec((BR1, SLOTS), lambda i: (i, 0)),
        ],
        out_shape=[
            jax.ShapeDtypeStruct((N, SLOTS), jnp.int32),
            jax.ShapeDtypeStruct((N, SLOTS), jnp.float32),
        ],
        scratch_shapes=[pltpu.VMEM((BR1, N), jnp.float32)],
        compiler_params=pltpu.CompilerParams(
            dimension_semantics=("parallel",)),
    )(x, xT)


_GW = 128


def _gather(table, idx_flat):
    total = idx_flat.shape[1]
    mesh = plsc.VectorSubcoreMesh(core_axis_name="core",
                                  subcore_axis_name="subcore")

    @pl.kernel(
        out_type=jax.ShapeDtypeStruct((total, table.shape[1]), table.dtype),
        mesh=mesh)
    def kern(tab_hbm, i_hbm, o_hbm):
        def body(i_vmem, o_vmem):
            pltpu.sync_copy(tab_hbm.at[i_vmem.at[0]], o_vmem)

        pltpu.emit_pipeline(
            body,
            grid=(total // _GW,),
            in_specs=[pl.BlockSpec((1, _GW), index_map=lambda i: (0, i))],
            out_specs=[pl.BlockSpec((_GW, table.shape[1]),
                                    index_map=lambda i: (i, 0))],
            core_axis_name=("core", "subcore"),
            dimension_semantics=(pltpu.PARALLEL,),
        )(i_hbm, o_hbm)

    return kern(table, idx_flat)


def _layer1_body(x_ref, xg_ref, mb_ref, W1l_ref, W1r_ref, A1_ref, b1_ref,
                 a1_ref, W2l_ref, W2r_ref, yl_ref, yr_ref):
    xr = jnp.dot(x_ref[...], W1r_ref[...],
                 preferred_element_type=jnp.float32)
    alphas = []
    for s in range(SLOTS):
        xlg = jnp.dot(xg_ref[s], W1l_ref[...],
                      preferred_element_type=jnp.float32)
        e = _leaky(xlg + xr)
        al = jnp.dot(e, A1_ref[...],
                     preferred_element_type=jnp.float32)
        alphas.append(al + mb_ref[:, s:s + 1])
    m = alphas[0]
    for s in range(1, SLOTS):
        m = jnp.maximum(m, alphas[s])
    exs = [jnp.exp(al - m) for al in alphas]
    den = exs[0]
    for s in range(1, SLOTS):
        den = den + exs[s]
    acc = jnp.zeros((BR2, HEADS * REP), jnp.float32)
    for s in range(SLOTS):
        w = exs[s] / den
        wf = jnp.broadcast_to(w[:, :, None], (BR2, HEADS, REP))
        xlg = jnp.dot(xg_ref[s], W1l_ref[...],
                      preferred_element_type=jnp.float32)
        acc = acc + wf.reshape(BR2, HEADS * REP) * xlg
    h = acc + b1_ref[...]
    h = jnp.where(h > 0, h, a1_ref[0, 0] * h)
    yl_ref[...] = jnp.dot(h, W2l_ref[...],
                          preferred_element_type=jnp.float32)
    yr_ref[...] = jnp.dot(h, W2r_ref[...],
                          preferred_element_type=jnp.float32)


def _layer1(x, xg, mb, W1l, W1r, A1, b1, a1, W2l, W2r):
    D1 = HEADS * REP
    return pl.pallas_call(
        _layer1_body,
        grid=(N // BR2,),
        in_specs=[
            pl.BlockSpec((BR2, IN_CH), lambda i: (i, 0)),
            pl.BlockSpec((SLOTS, BR2, IN_CH), lambda i: (0, i, 0)),
            pl.BlockSpec((BR2, SLOTS), lambda i: (i, 0)),
            pl.BlockSpec((IN_CH, D1), lambda i: (0, 0)),
            pl.BlockSpec((IN_CH, D1), lambda i: (0, 0)),
            pl.BlockSpec((D1, HEADS), lambda i: (0, 0)),
            pl.BlockSpec((1, D1), lambda i: (0, 0)),
            pl.BlockSpec((1, 1), lambda i: (0, 0)),
            pl.BlockSpec((D1, REP), lambda i: (0, 0)),
            pl.BlockSpec((D1, REP), lambda i: (0, 0)),
        ],
        out_specs=[
            pl.BlockSpec((BR2, REP), lambda i: (i, 0)),
            pl.BlockSpec((BR2, REP), lambda i: (i, 0)),
        ],
        out_shape=[
            jax.ShapeDtypeStruct((N, REP), jnp.float32),
            jax.ShapeDtypeStruct((N, REP), jnp.float32),
        ],
        compiler_params=pltpu.CompilerParams(
            dimension_semantics=("parallel",)),
    )(x, xg, mb, W1l, W1r, A1, b1, a1, W2l, W2r)


def _layer2_body(ylg_ref, yr_ref, mb_ref, A2_ref, b2_ref, a2_ref, o_ref):
    yr = yr_ref[...]
    alphas = []
    for s in range(SLOTS):
        e = _leaky(ylg_ref[s] + yr)
        al = jnp.dot(e, A2_ref[...],
                     preferred_element_type=jnp.float32)
        alphas.append(al[:, :1] + mb_ref[:, s:s + 1])
    m = alphas[0]
    for s in range(1, SLOTS):
        m = jnp.maximum(m, alphas[s])
    exs = [jnp.exp(al - m) for al in alphas]
    den = exs[0]
    for s in range(1, SLOTS):
        den = den + exs[s]
    acc = jnp.zeros((BR4, REP), jnp.float32)
    for s in range(SLOTS):
        acc = acc + (exs[s] / den) * ylg_ref[s]
    out = acc + b2_ref[...]
    o_ref[...] = jnp.where(out > 0, out, a2_ref[0, 0] * out)


def _layer2(ylg, yr, mb, A2, b2, a2):
    return pl.pallas_call(
        _layer2_body,
        grid=(N // BR4,),
        in_specs=[
            pl.BlockSpec((SLOTS, BR4, REP), lambda i: (0, i, 0)),
            pl.BlockSpec((BR4, REP), lambda i: (i, 0)),
            pl.BlockSpec((BR4, SLOTS), lambda i: (i, 0)),
            pl.BlockSpec((REP, SLOTS), lambda i: (0, 0)),
            pl.BlockSpec((1, REP), lambda i: (0, 0)),
            pl.BlockSpec((1, 1), lambda i: (0, 0)),
        ],
        out_specs=pl.BlockSpec((BR4, REP), lambda i: (i, 0)),
        out_shape=jax.ShapeDtypeStruct((N, REP), jnp.float32),
        compiler_params=pltpu.CompilerParams(
            dimension_semantics=("parallel",)),
    )(ylg, yr, mb, A2, b2, a2)


def kernel(x, W1l, W1r, att1, b1, a1, W2l, W2r, att2, b2, a2):
    xT = x.T
    A1 = (jnp.eye(HEADS, dtype=jnp.float32)[:, None, :]
          * att1[:, :, None]).reshape(HEADS * REP, HEADS)
    A2 = jnp.pad(att2.reshape(REP, 1), ((0, 0), (0, SLOTS - 1)))
    nbr, mb = _topk(x, xT)
    idx = nbr.T.reshape(1, SLOTS * N)
    xg = _gather(x, idx).reshape(SLOTS, N, IN_CH)
    yl, yr = _layer1(x, xg, mb, W1l, W1r, A1,
                     b1.reshape(1, HEADS * REP),
                     jnp.reshape(a1, (1, 1)),
                     W2l, W2r)
    ylg = _gather(yl, idx).reshape(SLOTS, N, REP)
    out = _layer2(ylg, yr, mb, A2,
                  b2.reshape(1, REP),
                  jnp.reshape(a2, (1, 1)))
    return out
